# baseline (speedup 1.0000x reference)
import functools

import jax
import jax.numpy as jnp
from jax import lax
from jax.experimental import pallas as pl
from jax.experimental.pallas import tpu as pltpu
from jax.experimental.pallas import tpu_sc as plsc


def _aff_body(dn_ref, sp_ref, sc_ref, out_ref):
    dn = dn_ref[0, 0]
    sp = sp_ref[0]
    sc = sc_ref[0]
    M, TN = sp.shape[0], dn.shape[1]
    aa = jnp.sum(dn * dn, axis=0, keepdims=True)
    bb = jnp.sum(sp * sp, axis=1, keepdims=True)
    ab = jax.lax.dot_general(sp, dn, (((1,), (0,)), ((), ())),
                             preferred_element_type=jnp.float32)
    d2 = (aa + bb) - 2.0 * ab
    m1 = jnp.min(d2, axis=0, keepdims=True)
    eq1 = d2 == m1
    d2b = jnp.where(eq1, jnp.inf, d2)
    m2 = jnp.min(d2b, axis=0, keepdims=True)
    sel = (eq1 | (d2b == m2)).astype(jnp.float32)
    aff = jax.lax.dot_general(sc, sel, (((0,), (0,)), ((), ())),
                              preferred_element_type=jnp.float32)
    out_ref[0, 0] = aff * 0.5


def _view_body(ad_ref, tv_ref, nvs_ref, idx_ref, val_ref):
    tv = tv_ref[...]
    V = tv.shape[1]
    bbv = jnp.sum(tv * tv, axis=0, keepdims=True)
    nvs = nvs_ref[0]
    M = nvs.shape[0]
    iota = jax.lax.broadcasted_iota(jnp.int32, (M, V), 1)
    g = []
    for j in range(3):
        dj = ad_ref[0, j]
        aaj = jnp.sum(dj * dj, axis=1, keepdims=True)
        dots = jax.lax.dot_general(dj, tv, (((1,), (0,)), ((), ())),
                                   preferred_element_type=jnp.float32)
        q = (aaj + bbv) - 2.0 * dots
        m = jnp.min(q, axis=1, keepdims=True)
        thr = m + jnp.abs(m) * 3e-7
        wnd = q <= thr
        c1 = jnp.min(jnp.where(wnd, iota, V), axis=1, keepdims=True)
        c2 = jnp.min(jnp.where(wnd & (iota != c1), iota, V),
                     axis=1, keepdims=True)
        qc1 = jnp.max(jnp.where(iota == c1, q, -jnp.inf),
                      axis=1, keepdims=True)
        tie1 = (jnp.sqrt(jnp.maximum(qc1, 0.0))
                == jnp.sqrt(jnp.maximum(m, 0.0)))
        vi = jnp.where(tie1, c1, jnp.minimum(c2, V - 1))
        g.append(vi)
    nv0, nv1, nv2 = (nvs[:, j:j + 1] for j in range(3))
    v0 = jnp.where(g[0] == g[2], nv2, jnp.where(g[0] == g[1], nv1, nv0))
    v1 = jnp.where(g[1] == g[2], nv2, nv1)
    idx_ref[0] = jnp.concatenate(g, axis=1)
    val_ref[0] = jnp.concatenate([v0, v1, nv2], axis=1)


def _make_sc_scatter(B, M, V):
    info = plsc.get_sparse_core_info()
    nw = info.num_cores * info.num_subcores
    n_entries = B * M * 3
    epw = n_entries // nw
    rows_pw = epw // 3
    n_chunks = 2
    rows_pc = rows_pw // n_chunks
    epc = epw // n_chunks
    mesh = plsc.VectorSubcoreMesh(core_axis_name="c", subcore_axis_name="s")

    @functools.partial(
        pl.kernel, mesh=mesh,
        out_type=jax.ShapeDtypeStruct((B, M, V), jnp.float32),
        compiler_params=pltpu.CompilerParams(needs_layout_passes=False),
        scratch_types=[
            pltpu.VMEM((epw,), jnp.int32),
            pltpu.VMEM((epw,), jnp.float32),
            pltpu.VMEM((rows_pc, V), jnp.float32),
        ],
    )
    def sc_scatter(idx_hbm, val_hbm, out_hbm, idx_v, val_v, buf_v):
        wid = lax.axis_index("s") * info.num_cores + lax.axis_index("c")
        ebase = wid * epw
        pltpu.sync_copy(idx_hbm.at[pl.ds(ebase, epw)], idx_v)
        pltpu.sync_copy(val_hbm.at[pl.ds(ebase, epw)], val_v)
        zero16 = jnp.zeros((16,), jnp.float32)
        lane = lax.iota(jnp.int32, 16)

        def zrow(r, _):
            for u in range(V // 16):
                buf_v[r, pl.ds(u * 16, 16)] = zero16
            return 0

        lax.fori_loop(0, rows_pc, zrow, 0)
        for chunk in range(n_chunks):
            row_base = wid * rows_pw + chunk * rows_pc
            for grp in range(epc // 16):
                off = chunk * epc + grp * 16
                lrow = (ebase + off + lane) // 3 - row_base
                lcol = idx_v[pl.ds(off, 16)]
                plsc.store_scatter(buf_v, [lrow, lcol], val_v[pl.ds(off, 16)])
            bb = row_base // M
            mr = row_base - bb * M
            pltpu.sync_copy(buf_v, out_hbm.at[bb, pl.ds(mr, rows_pc)])
            if chunk + 1 < n_chunks:
                for grp in range(epc // 16):
                    off = chunk * epc + grp * 16
                    lrow = (ebase + off + lane) // 3 - row_base
                    lcol = idx_v[pl.ds(off, 16)]
                    plsc.store_scatter(buf_v, [lrow, lcol], zero16)

    return sc_scatter


def kernel(dense_points, sparse_points, normalized_scores,
           approach_directions, normalized_view_score, template_views):
    B, N, _ = dense_points.shape
    M = sparse_points.shape[1]
    V = template_views.shape[0]
    NT = 10
    TN = N // NT

    dnT = dense_points.reshape(B, NT, TN, 3).transpose(0, 1, 3, 2)
    sc2 = normalized_scores[:, :, None]
    adT = approach_directions.transpose(0, 2, 1, 3)
    tvT = template_views.T

    sidx, sval = pl.pallas_call(
        _view_body,
        grid=(B,),
        in_specs=[
            pl.BlockSpec((1, 3, M, 3), lambda b: (b, 0, 0, 0)),
            pl.BlockSpec((3, V), lambda b: (0, 0)),
            pl.BlockSpec((1, M, 3), lambda b: (b, 0, 0)),
        ],
        out_specs=[
            pl.BlockSpec((1, M, 3), lambda b: (b, 0, 0)),
            pl.BlockSpec((1, M, 3), lambda b: (b, 0, 0)),
        ],
        out_shape=[
            jax.ShapeDtypeStruct((B, M, 3), jnp.int32),
            jax.ShapeDtypeStruct((B, M, 3), jnp.float32),
        ],
    )(adT, tvT, normalized_view_score)

    sc_scatter = _make_sc_scatter(B, M, V)
    vs = sc_scatter(sidx.reshape(-1), sval.reshape(-1))

    aff4 = pl.pallas_call(
        _aff_body,
        grid=(B, NT),
        in_specs=[
            pl.BlockSpec((1, 1, 3, TN), lambda b, t: (b, t, 0, 0)),
            pl.BlockSpec((1, M, 3), lambda b, t: (b, 0, 0)),
            pl.BlockSpec((1, M, 1), lambda b, t: (b, 0, 0)),
        ],
        out_specs=pl.BlockSpec((1, 1, 1, TN), lambda b, t: (b, t, 0, 0)),
        out_shape=jax.ShapeDtypeStruct((B, NT, 1, TN), jnp.float32),
    )(dnT, sparse_points, sc2)
    aff = aff4.reshape(B, N)

    return aff, vs


# speedup vs baseline: 1.1668x; 1.1668x over previous
import functools

import jax
import jax.numpy as jnp
from jax import lax
from jax.experimental import pallas as pl
from jax.experimental.pallas import tpu as pltpu
from jax.experimental.pallas import tpu_sc as plsc


def _aff_body(dn_ref, sp_ref, sc_ref, out_ref):
    dn = dn_ref[0, 0]
    sp = sp_ref[0]
    sc = sc_ref[0]
    M, TN = sp.shape[0], dn.shape[1]
    aa = jnp.sum(dn * dn, axis=0, keepdims=True)
    bb = jnp.sum(sp * sp, axis=1, keepdims=True)
    ab = jax.lax.dot_general(sp, dn, (((1,), (0,)), ((), ())),
                             preferred_element_type=jnp.float32)
    d2 = (aa + bb) - 2.0 * ab
    m1 = jnp.min(d2, axis=0, keepdims=True)
    d2b = jnp.where(d2 == m1, jnp.inf, d2)
    m2 = jnp.min(d2b, axis=0, keepdims=True)
    sel = (d2 <= m2).astype(jnp.float32)
    aff = jax.lax.dot_general(sc, sel, (((0,), (0,)), ((), ())),
                              preferred_element_type=jnp.float32)
    out_ref[0, 0] = aff * 0.5


def _view_body(ad_ref, tv_ref, nvs_ref, idx_ref, val_ref):
    tv = tv_ref[...]
    V = tv.shape[1]
    bbv = jnp.sum(tv * tv, axis=0, keepdims=True)
    nvs = nvs_ref[0]
    M = nvs.shape[0]
    iota = jax.lax.broadcasted_iota(jnp.int32, (M, V), 1)
    g = []
    for j in range(3):
        dj = ad_ref[0, j]
        aaj = jnp.sum(dj * dj, axis=1, keepdims=True)
        dots = jax.lax.dot_general(dj, tv, (((1,), (0,)), ((), ())),
                                   preferred_element_type=jnp.float32)
        dv = jnp.sqrt(jnp.maximum((aaj + bbv) - 2.0 * dots, 0.0))
        mv = jnp.min(dv, axis=1, keepdims=True)
        vi = jnp.min(jnp.where(dv == mv, iota, V), axis=1, keepdims=True)
        g.append(vi)
    nv0, nv1, nv2 = (nvs[:, j:j + 1] for j in range(3))
    v0 = jnp.where(g[0] == g[2], nv2, jnp.where(g[0] == g[1], nv1, nv0))
    v1 = jnp.where(g[1] == g[2], nv2, nv1)
    idx_ref[0] = jnp.concatenate(g, axis=1)
    val_ref[0] = jnp.concatenate([v0, v1, nv2], axis=1)


def _make_sc_scatter(B, M, V):
    info = plsc.get_sparse_core_info()
    nw = info.num_cores * info.num_subcores
    n_entries = B * M * 3
    epw = n_entries // nw
    rows_pw = epw // 3
    n_chunks = 2
    rows_pc = rows_pw // n_chunks
    epc = epw // n_chunks
    mesh = plsc.VectorSubcoreMesh(core_axis_name="c", subcore_axis_name="s")

    @functools.partial(
        pl.kernel, mesh=mesh,
        out_type=jax.ShapeDtypeStruct((B, M, V), jnp.float32),
        compiler_params=pltpu.CompilerParams(needs_layout_passes=False),
        scratch_types=[
            pltpu.VMEM((epw,), jnp.int32),
            pltpu.VMEM((epw,), jnp.float32),
            pltpu.VMEM((rows_pc, V), jnp.float32),
        ],
    )
    def sc_scatter(idx_hbm, val_hbm, out_hbm, idx_v, val_v, buf_v):
        wid = lax.axis_index("s") * info.num_cores + lax.axis_index("c")
        ebase = wid * epw
        pltpu.sync_copy(idx_hbm.at[pl.ds(ebase, epw)], idx_v)
        pltpu.sync_copy(val_hbm.at[pl.ds(ebase, epw)], val_v)
        zero16 = jnp.zeros((16,), jnp.float32)
        lane = lax.iota(jnp.int32, 16)

        def zrow(r, _):
            for u in range(V // 16):
                buf_v[r, pl.ds(u * 16, 16)] = zero16
            return 0

        lax.fori_loop(0, rows_pc, zrow, 0)
        for chunk in range(n_chunks):
            row_base = wid * rows_pw + chunk * rows_pc
            for grp in range(epc // 16):
                off = chunk * epc + grp * 16
                lrow = (ebase + off + lane) // 3 - row_base
                lcol = idx_v[pl.ds(off, 16)]
                plsc.store_scatter(buf_v, [lrow, lcol], val_v[pl.ds(off, 16)])
            bb = row_base // M
            mr = row_base - bb * M
            pltpu.sync_copy(buf_v, out_hbm.at[bb, pl.ds(mr, rows_pc)])
            if chunk + 1 < n_chunks:
                for grp in range(epc // 16):
                    off = chunk * epc + grp * 16
                    lrow = (ebase + off + lane) // 3 - row_base
                    lcol = idx_v[pl.ds(off, 16)]
                    plsc.store_scatter(buf_v, [lrow, lcol], zero16)

    return sc_scatter


def kernel(dense_points, sparse_points, normalized_scores,
           approach_directions, normalized_view_score, template_views):
    B, N, _ = dense_points.shape
    M = sparse_points.shape[1]
    V = template_views.shape[0]
    NT = 8
    TN = N // NT

    dnT = dense_points.reshape(B, NT, TN, 3).transpose(0, 1, 3, 2)
    sc2 = normalized_scores[:, :, None]
    adT = approach_directions.transpose(0, 2, 1, 3)
    tvT = template_views.T

    sidx, sval = pl.pallas_call(
        _view_body,
        grid=(B,),
        in_specs=[
            pl.BlockSpec((1, 3, M, 3), lambda b: (b, 0, 0, 0)),
            pl.BlockSpec((3, V), lambda b: (0, 0)),
            pl.BlockSpec((1, M, 3), lambda b: (b, 0, 0)),
        ],
        out_specs=[
            pl.BlockSpec((1, M, 3), lambda b: (b, 0, 0)),
            pl.BlockSpec((1, M, 3), lambda b: (b, 0, 0)),
        ],
        out_shape=[
            jax.ShapeDtypeStruct((B, M, 3), jnp.int32),
            jax.ShapeDtypeStruct((B, M, 3), jnp.float32),
        ],
    )(adT, tvT, normalized_view_score)

    sc_scatter = _make_sc_scatter(B, M, V)
    vs = sc_scatter(sidx.reshape(-1), sval.reshape(-1))

    aff4 = pl.pallas_call(
        _aff_body,
        grid=(B, NT),
        in_specs=[
            pl.BlockSpec((1, 1, 3, TN), lambda b, t: (b, t, 0, 0)),
            pl.BlockSpec((1, M, 3), lambda b, t: (b, 0, 0)),
            pl.BlockSpec((1, M, 1), lambda b, t: (b, 0, 0)),
        ],
        out_specs=pl.BlockSpec((1, 1, 1, TN), lambda b, t: (b, t, 0, 0)),
        out_shape=jax.ShapeDtypeStruct((B, NT, 1, TN), jnp.float32),
    )(dnT, sparse_points, sc2)
    aff = aff4.reshape(B, N)

    return aff, vs


# speedup vs baseline: 1.1919x; 1.0216x over previous
import functools

import jax
import jax.numpy as jnp
from jax import lax
from jax.experimental import pallas as pl
from jax.experimental.pallas import tpu as pltpu
from jax.experimental.pallas import tpu_sc as plsc


def _aff_body(dn_ref, sp_ref, sc_ref, out_ref):
    dn = dn_ref[0, 0]
    sp = sp_ref[0]
    sc = sc_ref[0]
    M, TN = sp.shape[0], dn.shape[1]
    aa = jnp.sum(dn * dn, axis=0, keepdims=True)
    bb = jnp.sum(sp * sp, axis=1, keepdims=True)
    ab = jax.lax.dot_general(sp, dn, (((1,), (0,)), ((), ())),
                             preferred_element_type=jnp.float32)
    d2 = (aa + bb) - 2.0 * ab
    m1 = jnp.min(d2, axis=0, keepdims=True)
    d2b = jnp.where(d2 == m1, jnp.inf, d2)
    m2 = jnp.min(d2b, axis=0, keepdims=True)
    sel = (d2 <= m2).astype(jnp.float32)
    aff = jax.lax.dot_general(sc, sel, (((0,), (0,)), ((), ())),
                              preferred_element_type=jnp.float32)
    out_ref[0, 0] = aff * 0.5


def _view_body(ad_ref, tv_ref, nvs_ref, idx_ref, val_ref):
    tv = tv_ref[...]
    V = tv.shape[1]
    bbv = jnp.sum(tv * tv, axis=0, keepdims=True)
    nvs = nvs_ref[0]
    M = nvs.shape[0]
    iota = jax.lax.broadcasted_iota(jnp.int32, (M, V), 1)
    g = []
    for j in range(3):
        dj = ad_ref[0, j]
        aaj = jnp.sum(dj * dj, axis=1, keepdims=True)
        dots = jax.lax.dot_general(dj, tv, (((1,), (0,)), ((), ())),
                                   preferred_element_type=jnp.float32)
        dv = jnp.sqrt(jnp.maximum((aaj + bbv) - 2.0 * dots, 0.0))
        mv = jnp.min(dv, axis=1, keepdims=True)
        vi = jnp.min(jnp.where(dv == mv, iota, V), axis=1, keepdims=True)
        g.append(vi)
    nv0, nv1, nv2 = (nvs[:, j:j + 1] for j in range(3))
    v0 = jnp.where(g[0] == g[2], nv2, jnp.where(g[0] == g[1], nv1, nv0))
    v1 = jnp.where(g[1] == g[2], nv2, nv1)
    idx_ref[0] = jnp.concatenate(g, axis=1)
    val_ref[0] = jnp.concatenate([v0, v1, nv2], axis=1)


def _make_sc_scatter(B, M, V):
    info = plsc.get_sparse_core_info()
    nw = info.num_cores * info.num_subcores
    n_entries = B * M * 3
    epw = n_entries // nw
    rows_pw = epw // 3
    n_chunks = 2
    rows_pc = rows_pw // n_chunks
    epc = epw // n_chunks
    mesh = plsc.VectorSubcoreMesh(core_axis_name="c", subcore_axis_name="s")

    @functools.partial(
        pl.kernel, mesh=mesh,
        out_type=jax.ShapeDtypeStruct((B, M, V), jnp.float32),
        compiler_params=pltpu.CompilerParams(needs_layout_passes=False),
        scratch_types=[
            pltpu.VMEM((epw,), jnp.int32),
            pltpu.VMEM((epw,), jnp.float32),
            pltpu.VMEM((rows_pc, V), jnp.float32),
        ],
    )
    def sc_scatter(idx_hbm, val_hbm, out_hbm, idx_v, val_v, buf_v):
        wid = lax.axis_index("s") * info.num_cores + lax.axis_index("c")
        ebase = wid * epw
        pltpu.sync_copy(idx_hbm.at[pl.ds(ebase, epw)], idx_v)
        pltpu.sync_copy(val_hbm.at[pl.ds(ebase, epw)], val_v)
        zero16 = jnp.zeros((16,), jnp.float32)
        lane = lax.iota(jnp.int32, 16)

        def zrow(r, _):
            for u in range(V // 16):
                buf_v[r, pl.ds(u * 16, 16)] = zero16
            return 0

        lax.fori_loop(0, rows_pc, zrow, 0)
        for chunk in range(n_chunks):
            row_base = wid * rows_pw + chunk * rows_pc
            for grp in range(epc // 16):
                off = chunk * epc + grp * 16
                lrow = (ebase + off + lane) // 3 - row_base
                lcol = idx_v[pl.ds(off, 16)]
                plsc.store_scatter(buf_v, [lrow, lcol], val_v[pl.ds(off, 16)])
            bb = row_base // M
            mr = row_base - bb * M
            pltpu.sync_copy(buf_v, out_hbm.at[bb, pl.ds(mr, rows_pc)])
            if chunk + 1 < n_chunks:
                for grp in range(epc // 16):
                    off = chunk * epc + grp * 16
                    lrow = (ebase + off + lane) // 3 - row_base
                    lcol = idx_v[pl.ds(off, 16)]
                    plsc.store_scatter(buf_v, [lrow, lcol], zero16)

    return sc_scatter


def kernel(dense_points, sparse_points, normalized_scores,
           approach_directions, normalized_view_score, template_views):
    B, N, _ = dense_points.shape
    M = sparse_points.shape[1]
    V = template_views.shape[0]
    NT = 5
    TN = N // NT

    dnT = dense_points.reshape(B, NT, TN, 3).transpose(0, 1, 3, 2)
    sc2 = normalized_scores[:, :, None]
    adT = approach_directions.transpose(0, 2, 1, 3)
    tvT = template_views.T

    sidx, sval = pl.pallas_call(
        _view_body,
        grid=(B,),
        in_specs=[
            pl.BlockSpec((1, 3, M, 3), lambda b: (b, 0, 0, 0)),
            pl.BlockSpec((3, V), lambda b: (0, 0)),
            pl.BlockSpec((1, M, 3), lambda b: (b, 0, 0)),
        ],
        out_specs=[
            pl.BlockSpec((1, M, 3), lambda b: (b, 0, 0)),
            pl.BlockSpec((1, M, 3), lambda b: (b, 0, 0)),
        ],
        out_shape=[
            jax.ShapeDtypeStruct((B, M, 3), jnp.int32),
            jax.ShapeDtypeStruct((B, M, 3), jnp.float32),
        ],
    )(adT, tvT, normalized_view_score)

    sc_scatter = _make_sc_scatter(B, M, V)
    vs = sc_scatter(sidx.reshape(-1), sval.reshape(-1))

    aff4 = pl.pallas_call(
        _aff_body,
        grid=(B, NT),
        in_specs=[
            pl.BlockSpec((1, 1, 3, TN), lambda b, t: (b, t, 0, 0)),
            pl.BlockSpec((1, M, 3), lambda b, t: (b, 0, 0)),
            pl.BlockSpec((1, M, 1), lambda b, t: (b, 0, 0)),
        ],
        out_specs=pl.BlockSpec((1, 1, 1, TN), lambda b, t: (b, t, 0, 0)),
        out_shape=jax.ShapeDtypeStruct((B, NT, 1, TN), jnp.float32),
    )(dnT, sparse_points, sc2)
    aff = aff4.reshape(B, N)

    return aff, vs


# speedup vs baseline: 1.1924x; 1.0004x over previous
import functools

import jax
import jax.numpy as jnp
from jax import lax
from jax.experimental import pallas as pl
from jax.experimental.pallas import tpu as pltpu
from jax.experimental.pallas import tpu_sc as plsc


def _aff_body(dn_ref, sp_ref, sc_ref, out_ref):
    dn = dn_ref[0, 0]
    sp = sp_ref[0]
    sc = sc_ref[0]
    M, TN = sp.shape[0], dn.shape[1]
    aa = jnp.sum(dn * dn, axis=0, keepdims=True)
    bb = jnp.sum(sp * sp, axis=1, keepdims=True)
    ab = jax.lax.dot_general(sp, dn, (((1,), (0,)), ((), ())),
                             preferred_element_type=jnp.float32)
    d2 = (aa + bb) - 2.0 * ab
    m1 = jnp.min(d2, axis=0, keepdims=True)
    d2b = jnp.where(d2 == m1, jnp.inf, d2)
    m2 = jnp.min(d2b, axis=0, keepdims=True)
    sel = (d2 <= m2).astype(jnp.float32)
    aff = jax.lax.dot_general(sc, sel, (((0,), (0,)), ((), ())),
                              preferred_element_type=jnp.float32)
    out_ref[0, 0] = aff * 0.5


def _view_body(ad_ref, tv_ref, nvs_ref, idx_ref, val_ref):
    tv = tv_ref[...]
    V = tv.shape[1]
    bbv = jnp.sum(tv * tv, axis=0, keepdims=True)
    nvs = nvs_ref[0]
    M = nvs.shape[0]
    iota = jax.lax.broadcasted_iota(jnp.int32, (M, V), 1)
    g = []
    for j in range(3):
        dj = ad_ref[0, j]
        aaj = jnp.sum(dj * dj, axis=1, keepdims=True)
        dots = jax.lax.dot_general(dj, tv, (((1,), (0,)), ((), ())),
                                   preferred_element_type=jnp.float32)
        dv = jnp.sqrt(jnp.maximum((aaj + bbv) - 2.0 * dots, 0.0))
        mv = jnp.min(dv, axis=1, keepdims=True)
        vi = jnp.min(jnp.where(dv == mv, iota, V), axis=1, keepdims=True)
        g.append(vi)
    nv0, nv1, nv2 = (nvs[:, j:j + 1] for j in range(3))
    v0 = jnp.where(g[0] == g[2], nv2, jnp.where(g[0] == g[1], nv1, nv0))
    v1 = jnp.where(g[1] == g[2], nv2, nv1)
    idx_ref[0] = jnp.concatenate(g, axis=1)
    val_ref[0] = jnp.concatenate([v0, v1, nv2], axis=1)


def _make_sc_scatter(B, M, V):
    info = plsc.get_sparse_core_info()
    nw = info.num_cores * info.num_subcores
    n_entries = B * M * 3
    epw = n_entries // nw
    rows_pw = epw // 3
    n_chunks = 2
    rows_pc = rows_pw // n_chunks
    epc = epw // n_chunks
    mesh = plsc.VectorSubcoreMesh(core_axis_name="c", subcore_axis_name="s")

    @functools.partial(
        pl.kernel, mesh=mesh,
        out_type=jax.ShapeDtypeStruct((B, M, V), jnp.float32),
        compiler_params=pltpu.CompilerParams(needs_layout_passes=False),
        scratch_types=[
            pltpu.VMEM((epw,), jnp.int32),
            pltpu.VMEM((epw,), jnp.float32),
            pltpu.VMEM((rows_pc, V), jnp.float32),
        ],
    )
    def sc_scatter(idx_hbm, val_hbm, out_hbm, idx_v, val_v, buf_v):
        wid = lax.axis_index("s") * info.num_cores + lax.axis_index("c")
        ebase = wid * epw
        pltpu.sync_copy(idx_hbm.at[pl.ds(ebase, epw)], idx_v)
        pltpu.sync_copy(val_hbm.at[pl.ds(ebase, epw)], val_v)
        zero16 = jnp.zeros((16,), jnp.float32)
        lane = lax.iota(jnp.int32, 16)

        def zrow(r, _):
            for u in range(V // 16):
                buf_v[r, pl.ds(u * 16, 16)] = zero16
            return 0

        lax.fori_loop(0, rows_pc, zrow, 0)
        for chunk in range(n_chunks):
            row_base = wid * rows_pw + chunk * rows_pc
            for grp in range(epc // 16):
                off = chunk * epc + grp * 16
                lrow = (ebase + off + lane) // 3 - row_base
                lcol = idx_v[pl.ds(off, 16)]
                plsc.store_scatter(buf_v, [lrow, lcol], val_v[pl.ds(off, 16)])
            bb = row_base // M
            mr = row_base - bb * M
            pltpu.sync_copy(buf_v, out_hbm.at[bb, pl.ds(mr, rows_pc)])
            if chunk + 1 < n_chunks:
                for grp in range(epc // 16):
                    off = chunk * epc + grp * 16
                    lrow = (ebase + off + lane) // 3 - row_base
                    lcol = idx_v[pl.ds(off, 16)]
                    plsc.store_scatter(buf_v, [lrow, lcol], zero16)

    return sc_scatter


def kernel(dense_points, sparse_points, normalized_scores,
           approach_directions, normalized_view_score, template_views):
    B, N, _ = dense_points.shape
    M = sparse_points.shape[1]
    V = template_views.shape[0]
    NT = 5
    TN = N // NT

    dnT = dense_points.reshape(B, NT, TN, 3).transpose(0, 1, 3, 2)
    sc2 = normalized_scores[:, :, None]
    adT = approach_directions.transpose(0, 2, 1, 3)
    tvT = template_views.T

    sidx, sval = pl.pallas_call(
        _view_body,
        grid=(B,),
        in_specs=[
            pl.BlockSpec((1, 3, M, 3), lambda b: (b, 0, 0, 0)),
            pl.BlockSpec((3, V), lambda b: (0, 0)),
            pl.BlockSpec((1, M, 3), lambda b: (b, 0, 0)),
        ],
        out_specs=[
            pl.BlockSpec((1, M, 3), lambda b: (b, 0, 0)),
            pl.BlockSpec((1, M, 3), lambda b: (b, 0, 0)),
        ],
        out_shape=[
            jax.ShapeDtypeStruct((B, M, 3), jnp.int32),
            jax.ShapeDtypeStruct((B, M, 3), jnp.float32),
        ],
        compiler_params=pltpu.CompilerParams(
            dimension_semantics=("parallel",)),
    )(adT, tvT, normalized_view_score)

    sc_scatter = _make_sc_scatter(B, M, V)
    vs = sc_scatter(sidx.reshape(-1), sval.reshape(-1))

    aff4 = pl.pallas_call(
        _aff_body,
        grid=(B, NT),
        in_specs=[
            pl.BlockSpec((1, 1, 3, TN), lambda b, t: (b, t, 0, 0)),
            pl.BlockSpec((1, M, 3), lambda b, t: (b, 0, 0)),
            pl.BlockSpec((1, M, 1), lambda b, t: (b, 0, 0)),
        ],
        out_specs=pl.BlockSpec((1, 1, 1, TN), lambda b, t: (b, t, 0, 0)),
        out_shape=jax.ShapeDtypeStruct((B, NT, 1, TN), jnp.float32),
        compiler_params=pltpu.CompilerParams(
            dimension_semantics=("parallel", "parallel")),
    )(dnT, sparse_points, sc2)
    aff = aff4.reshape(B, N)

    return aff, vs
